# baseline (speedup 1.0000x reference)
import jax
import jax.numpy as jnp
from jax.experimental import pallas as pl
from jax.experimental.pallas import tpu as pltpu


def _pool_kernel(x_ref, m_ref, w1_ref, w2_ref, o_ref, acc_ref, l_ref):
    j = pl.program_id(1)
    nj = pl.num_programs(1)
    a = w2_ref.shape[1]

    @pl.when(j == 0)
    def _init():
        acc_ref[...] = jnp.zeros_like(acc_ref)
        l_ref[...] = jnp.zeros_like(l_ref)

    w2 = w2_ref[...]
    bound = jnp.sum(jnp.abs(w2), axis=0, keepdims=True)
    x = x_ref[...]
    h = jnp.tanh(jnp.dot(x, w1_ref[...], preferred_element_type=jnp.float32))
    logits = jnp.dot(h, w2, preferred_element_type=jnp.float32) + (-bound)
    p = jnp.exp(logits) * m_ref[0]
    l_ref[...] += jnp.sum(p, axis=0, keepdims=True)
    acc_ref[...] += jax.lax.dot_general(p, x, (((0,), (0,)), ((), ())),
                                        preferred_element_type=jnp.float32)

    @pl.when(j == nj - 1)
    def _fin():
        winv = 1.0 / (l_ref[...] * float(a))
        o_ref[0] = jnp.dot(winv, acc_ref[...],
                           preferred_element_type=jnp.float32)


def kernel(data, padding_mask, W1, W2):
    B, S, H = data.shape
    U, A = W2.shape[0], W2.shape[1]
    nj = 4
    sb = S // nj
    data2 = data.reshape(B * S, H)
    mask3 = padding_mask.reshape(B, S, 1)
    out = pl.pallas_call(
        _pool_kernel,
        out_shape=jax.ShapeDtypeStruct((B, 1, H), jnp.float32),
        grid=(B, nj),
        in_specs=[
            pl.BlockSpec((sb, H), lambda b, j: (b * 4 + j, 0)),
            pl.BlockSpec((1, sb, 1), lambda b, j: (b, j, 0)),
            pl.BlockSpec((H, U), lambda b, j: (0, 0)),
            pl.BlockSpec((U, A), lambda b, j: (0, 0)),
        ],
        out_specs=pl.BlockSpec((1, 1, H), lambda b, j: (b, 0, 0)),
        scratch_shapes=[
            pltpu.VMEM((A, H), jnp.float32),
            pltpu.VMEM((1, A), jnp.float32),
        ],
        compiler_params=pltpu.CompilerParams(
            dimension_semantics=("parallel", "arbitrary"),
            vmem_limit_bytes=56 * 1024 * 1024,
        ),
        name="self_attn_pool",
    )(data2, mask3, W1, W2)
    return out.reshape(B, H)


# speedup vs baseline: 1.1417x; 1.1417x over previous
import jax
import jax.numpy as jnp
from jax.experimental import pallas as pl
from jax.experimental.pallas import tpu as pltpu


def _pool_kernel(x_ref, m_ref, w1_ref, w2_ref, o_ref, acc_ref, l_ref):
    j = pl.program_id(1)
    nj = pl.num_programs(1)
    a = w2_ref.shape[1]

    @pl.when(j == 0)
    def _init():
        acc_ref[...] = jnp.zeros_like(acc_ref)
        l_ref[...] = jnp.zeros_like(l_ref)

    w2 = w2_ref[...]
    bound = jnp.sum(jnp.abs(w2), axis=0, keepdims=True)
    x = x_ref[...]
    h = jnp.tanh(jnp.dot(x, w1_ref[...], preferred_element_type=jnp.float32))
    logits = jnp.dot(h, w2, preferred_element_type=jnp.float32) + (-bound)
    p = jnp.exp(logits) * m_ref[0]
    l_ref[...] += jnp.sum(p, axis=0, keepdims=True)
    acc_ref[...] += jax.lax.dot_general(p, x, (((0,), (0,)), ((), ())),
                                        preferred_element_type=jnp.float32)

    @pl.when(j == nj - 1)
    def _fin():
        winv = 1.0 / (l_ref[...] * float(a))
        o_ref[0] = jnp.dot(winv, acc_ref[...],
                           preferred_element_type=jnp.float32)


def kernel(data, padding_mask, W1, W2):
    B, S, H = data.shape
    U, A = W2.shape[0], W2.shape[1]
    nj = 2
    sb = S // nj
    data2 = data.reshape(B * S, H)
    mask3 = padding_mask.reshape(B, S, 1)
    out = pl.pallas_call(
        _pool_kernel,
        out_shape=jax.ShapeDtypeStruct((B, 1, H), jnp.float32),
        grid=(B, nj),
        in_specs=[
            pl.BlockSpec((sb, H), lambda b, j: (b * 2 + j, 0)),
            pl.BlockSpec((1, sb, 1), lambda b, j: (b, j, 0)),
            pl.BlockSpec((H, U), lambda b, j: (0, 0)),
            pl.BlockSpec((U, A), lambda b, j: (0, 0)),
        ],
        out_specs=pl.BlockSpec((1, 1, H), lambda b, j: (b, 0, 0)),
        scratch_shapes=[
            pltpu.VMEM((A, H), jnp.float32),
            pltpu.VMEM((1, A), jnp.float32),
        ],
        compiler_params=pltpu.CompilerParams(
            dimension_semantics=("parallel", "arbitrary"),
            vmem_limit_bytes=56 * 1024 * 1024,
        ),
        name="self_attn_pool",
    )(data2, mask3, W1, W2)
    return out.reshape(B, H)
